# baseline (speedup 1.0000x reference)
import functools

import jax
import jax.numpy as jnp
from jax import lax
from jax.experimental import pallas as pl
from jax.experimental.pallas import tpu as pltpu
from jax.experimental.pallas import tpu_sc as plsc

N = 10000
NP = 10240
E = 160000
EP = 172032
D = 256
DE = 16
H = 4
C = 256
HC = H * C
EPS = 1e-5
F32 = jnp.float32

NB = 16
BN = NP // NB
EB = 84
BE = EP // EB


def _leaky(x, s):
    return jnp.where(x >= 0, x, s * x)


def _embed_body(x_ref, w_ref, b_ref, g_ref, bb_ref, o_ref):
    h = jnp.dot(x_ref[...], w_ref[...].T, preferred_element_type=F32) + b_ref[...]
    mu = h.mean(-1, keepdims=True)
    var = ((h - mu) ** 2).mean(-1, keepdims=True)
    h = (h - mu) / jnp.sqrt(var + EPS) * g_ref[...] + bb_ref[...]
    o_ref[...] = _leaky(h, 0.01)


def _embed(xp, w, b, g, bb):
    return pl.pallas_call(
        _embed_body,
        grid=(NB,),
        in_specs=[
            pl.BlockSpec((BN, D), lambda i: (i, 0)),
            pl.BlockSpec((D, D), lambda i: (0, 0)),
            pl.BlockSpec((1, D), lambda i: (0, 0)),
            pl.BlockSpec((1, D), lambda i: (0, 0)),
            pl.BlockSpec((1, D), lambda i: (0, 0)),
        ],
        out_specs=pl.BlockSpec((BN, D), lambda i: (i, 0)),
        out_shape=jax.ShapeDtypeStruct((NP, D), F32),
    )(xp, w, b.reshape(1, D), g.reshape(1, D), bb.reshape(1, D))


def _lr_body(h_ref, wl_ref, bl_ref, wr_ref, br_ref, xl_ref, xr_ref, xlc_ref):
    h = h_ref[...]
    xl = jnp.dot(h, wl_ref[...].T, preferred_element_type=F32) + bl_ref[...]
    xr = jnp.dot(h, wr_ref[...].T, preferred_element_type=F32) + br_ref[...]
    xl_ref[...] = xl
    xr_ref[...] = xr
    for c in range(32):
        xlc_ref[c] = xl[:, c * 32:(c + 1) * 32]


def _lr_proj(h, wl, bl, wr, br):
    return pl.pallas_call(
        _lr_body,
        grid=(NB,),
        in_specs=[
            pl.BlockSpec((BN, D), lambda i: (i, 0)),
            pl.BlockSpec((HC, D), lambda i: (0, 0)),
            pl.BlockSpec((1, HC), lambda i: (0, 0)),
            pl.BlockSpec((HC, D), lambda i: (0, 0)),
            pl.BlockSpec((1, HC), lambda i: (0, 0)),
        ],
        out_specs=[
            pl.BlockSpec((BN, HC), lambda i: (i, 0)),
            pl.BlockSpec((BN, HC), lambda i: (i, 0)),
            pl.BlockSpec((32, BN, 32), lambda i: (0, i, 0)),
        ],
        out_shape=[
            jax.ShapeDtypeStruct((NP, HC), F32),
            jax.ShapeDtypeStruct((NP, HC), F32),
            jax.ShapeDtypeStruct((32, NP, 32), F32),
        ],
    )(h, wl, bl.reshape(1, HC), wr, br.reshape(1, HC))


def _ee_body(ea_ref, we_ref, o_ref):
    o_ref[...] = jnp.dot(ea_ref[...], we_ref[...].T, preferred_element_type=F32)


def _ee_proj(ea_p, we):
    return pl.pallas_call(
        _ee_body,
        grid=(EB,),
        in_specs=[
            pl.BlockSpec((BE, DE), lambda i: (i, 0)),
            pl.BlockSpec((HC, DE), lambda i: (0, 0)),
        ],
        out_specs=pl.BlockSpec((BE, HC), lambda i: (i, 0)),
        out_shape=jax.ShapeDtypeStruct((EP, HC), F32),
    )(ea_p, we)


def _recip_body(d_ref, o_ref):
    o_ref[...] = 1.0 / (d_ref[0] + d_ref[1] + 1e-16)


def _recip(den_part):
    return pl.pallas_call(
        _recip_body,
        grid=(1,),
        in_specs=[pl.BlockSpec((2, H, NP), lambda i: (0, 0, 0))],
        out_specs=pl.BlockSpec((H, NP), lambda i: (0, 0)),
        out_shape=jax.ShapeDtypeStruct((H, NP), F32),
    )(den_part)


def _stats_body(o_ref, cb_ref, p_ref):
    i = pl.program_id(0)
    rid = lax.broadcasted_iota(jnp.int32, (BN, 32), 0) + i * BN
    s = jnp.zeros((), F32)
    sq = jnp.zeros((), F32)
    for c in range(32):
        v = o_ref[c] + cb_ref[c]
        v = jnp.where(rid < N, v, 0.0)
        s = s + jnp.sum(v)
        sq = sq + jnp.sum(v * v)
    col = lax.broadcasted_iota(jnp.int32, (1, 1, 128), 2)
    p_ref[...] = jnp.where(col == 0, s, jnp.where(col == 1, sq, 0.0))


def _stats(out_cm, cb):
    return pl.pallas_call(
        _stats_body,
        grid=(NB,),
        in_specs=[
            pl.BlockSpec((32, BN, 32), lambda i: (0, i, 0)),
            pl.BlockSpec((32, 1, 32), lambda i: (0, 0, 0)),
        ],
        out_specs=pl.BlockSpec((1, 1, 128), lambda i: (i, 0, 0)),
        out_shape=jax.ShapeDtypeStruct((NB, 1, 128), F32),
    )(out_cm, cb.reshape(32, 1, 32))


def _post_body(o_ref, ps_ref, cb_ref, gg_ref, gb_ref, pw_ref, pb_ref,
               g_ref, b_ref, h_ref):
    cnt = float(N * HC)
    s = jnp.sum(ps_ref[:, :, 0])
    sq = jnp.sum(ps_ref[:, :, 1])
    mu = s / cnt
    var = sq / cnt - mu * mu
    std = jnp.sqrt(var)
    zs = []
    for c in range(32):
        v = o_ref[c] + cb_ref[c]
        z = (v - mu) / (std + EPS) * gg_ref[c] + gb_ref[c]
        zs.append(_leaky(z, 0.01))
    z = jnp.concatenate(zs, axis=-1)
    y = jnp.dot(z, pw_ref[...].T, preferred_element_type=F32) + pb_ref[...]
    mu2 = y.mean(-1, keepdims=True)
    var2 = ((y - mu2) ** 2).mean(-1, keepdims=True)
    y = (y - mu2) / jnp.sqrt(var2 + EPS) * g_ref[...] + b_ref[...]
    h_ref[...] = _leaky(y, 0.01)


def _post(out_cm, ps, cb, gg, gb, pw, pb, g, b):
    return pl.pallas_call(
        _post_body,
        grid=(NB,),
        in_specs=[
            pl.BlockSpec((32, BN, 32), lambda i: (0, i, 0)),
            pl.BlockSpec((NB, 1, 128), lambda i: (0, 0, 0)),
            pl.BlockSpec((32, 1, 32), lambda i: (0, 0, 0)),
            pl.BlockSpec((32, 1, 32), lambda i: (0, 0, 0)),
            pl.BlockSpec((32, 1, 32), lambda i: (0, 0, 0)),
            pl.BlockSpec((D, HC), lambda i: (0, 0)),
            pl.BlockSpec((1, D), lambda i: (0, 0)),
            pl.BlockSpec((1, D), lambda i: (0, 0)),
            pl.BlockSpec((1, D), lambda i: (0, 0)),
        ],
        out_specs=pl.BlockSpec((BN, D), lambda i: (i, 0)),
        out_shape=jax.ShapeDtypeStruct((NP, D), F32),
    )(out_cm, ps, cb.reshape(32, 1, 32), gg.reshape(32, 1, 32),
      gb.reshape(32, 1, 32), pw, pb.reshape(1, D), g.reshape(1, D),
      b.reshape(1, D))


def _max3_body(a_ref, b_ref, c_ref, o_ref):
    o_ref[...] = jnp.maximum(jnp.maximum(a_ref[...], b_ref[...]), c_ref[...])


def _max3(a, b, c):
    return pl.pallas_call(
        _max3_body,
        grid=(NB,),
        in_specs=[pl.BlockSpec((BN, D), lambda i: (i, 0))] * 3,
        out_specs=pl.BlockSpec((BN, D), lambda i: (i, 0)),
        out_shape=jax.ShapeDtypeStruct((NP, D), F32),
    )(a, b, c)


EPW = EP // 32
SBW = EPW // 128
ETB = EP // 16
RPT = ETB // 128
NPT = NP // 16


def _edge_a(xl, xr, ee, src16, dst16, attf):
    mesh = plsc.VectorSubcoreMesh(core_axis_name="c", subcore_axis_name="s")

    @functools.partial(
        pl.kernel,
        out_type=[
            jax.ShapeDtypeStruct((EP // 128, H, 128), F32),
            jax.ShapeDtypeStruct((2 * H * NP,), F32),
        ],
        mesh=mesh,
        scratch_types=[
            pltpu.VMEM((EP // 16 // 32, 16), jnp.int32),
            pltpu.VMEM((EP // 16 // 32, 16), jnp.int32),
            pltpu.VMEM((SBW, 128), jnp.int32),
            pltpu.VMEM((16, HC), F32),
            pltpu.VMEM((16, HC), F32),
            pltpu.VMEM((16, HC), F32),
            pltpu.VMEM((H, 128), F32),
            pltpu.VMEM((HC,), F32),
            pltpu.VMEM((NPT,), F32),
            pltpu.VMEM_SHARED((NP,), F32),
            pltpu.VMEM_SHARED((NP,), F32),
            pltpu.VMEM_SHARED((NP,), F32),
            pltpu.VMEM_SHARED((NP,), F32),
            pltpu.SemaphoreType.DMA,
            pltpu.SemaphoreType.DMA,
            pltpu.SemaphoreType.DMA,
        ],
        compiler_params=pltpu.CompilerParams(use_tc_tiling_on_sc=False, needs_layout_passes=False),
    )
    def k(xl_h, xr_h, ee_h, src_h, dst_h, att_h, ex_h, den_h,
          srcv, dstv, dstv128, xlb, xrb, eeb, exrow, attv, zbuf,
          d0, d1, d2, d3, s0, s1, s2):
        dens = (d0, d1, d2, d3)
        cid = lax.axis_index("c")
        sid = lax.axis_index("s")
        wid = sid * 2 + cid
        pltpu.sync_copy(src_h.at[wid], srcv)
        pltpu.sync_copy(dst_h.at[wid], dstv)
        pltpu.sync_copy(att_h, attv)

        def df(i, _):
            dstv128[i // 8, pl.ds((i % 8) * 16, 16)] = dstv[i]
            return 0
        lax.fori_loop(0, EP // 16 // 32, df, 0)

        def zf(i, _):
            zbuf[pl.ds(i * 16, 16)] = jnp.zeros((16,), F32)
            return 0
        lax.fori_loop(0, NPT // 16, zf, 0)
        for dh in dens:
            pltpu.sync_copy(zbuf, dh.at[pl.ds(sid * NPT, NPT)])
        plsc.subcore_barrier()

        def sb_loop(S, _):
            e0 = wid * EPW + S * 128

            def sb_body(sb, _):
                b = S * 8 + sb
                a1 = pltpu.async_copy(xl_h.at[srcv.at[b]], xlb, s0)
                a2 = pltpu.async_copy(xr_h.at[dstv.at[b]], xrb, s1)
                a3 = pltpu.async_copy(ee_h.at[pl.ds(e0 + sb * 16, 16)], eeb,
                                      s2)
                a1.wait()
                a2.wait()
                a3.wait()
                for h in range(H):
                    def cl(g, accs, h=h):
                        parts = list(accs)
                        row16 = lax.iota(jnp.int32, 16)
                        c0 = h * C + g * 16
                        att16 = attv[pl.ds(c0, 16)]
                        for jj in range(16):
                            colv = jnp.full((16,), jj, jnp.int32) + c0
                            xlv = plsc.load_gather(xlb, [row16, colv])
                            xrv = plsc.load_gather(xrb, [row16, colv])
                            eev = plsc.load_gather(eeb, [row16, colv])
                            z = xlv + xrv + eev
                            m = jnp.maximum(z, 0.2 * z)
                            parts[jj % 4] = parts[jj % 4] + att16[jj] * m
                        return tuple(parts)
                    z16 = jnp.zeros((16,), F32)
                    accs = lax.fori_loop(0, C // 16, cl, (z16, z16, z16, z16),
                                         unroll=2)
                    acc = (accs[0] + accs[1]) + (accs[2] + accs[3])
                    exrow[h, pl.ds(sb * 16, 16)] = jnp.exp(acc)
                return 0
            lax.fori_loop(0, 8, sb_body, 0)
            pltpu.sync_copy(exrow, ex_h.at[wid * SBW + S])
            for h in range(H):
                pltpu.sync_copy(exrow.at[h], dens[h].at[dstv128.at[S]],
                                add=True)
            return 0
        lax.fori_loop(0, SBW, sb_loop, 0)
        plsc.subcore_barrier()
        for h in range(H):
            off = cid * (H * NP) + h * NP + sid * NPT
            pltpu.sync_copy(dens[h].at[pl.ds(sid * NPT, NPT)],
                            den_h.at[pl.ds(off, NPT)])

    return k(xl, xr, ee, src16, dst16, attf)


def _edge_b(xlc_flat, src128, dst128, exq, r):
    mesh = plsc.VectorSubcoreMesh(core_axis_name="c", subcore_axis_name="s")

    @functools.partial(
        pl.kernel,
        out_type=jax.ShapeDtypeStruct((32, NP, 32), F32),
        mesh=mesh,
        scratch_types=[
            pltpu.VMEM((RPT, 128), jnp.int32),
            pltpu.VMEM((RPT, 128), jnp.int32),
            pltpu.VMEM((RPT, H, 128), F32),
            pltpu.VMEM((NP,), F32),
            pltpu.VMEM((RPT, 128), F32),
            pltpu.VMEM((RPT, 128), jnp.int32),
            pltpu.VMEM((128, 32), F32),
            pltpu.VMEM_SHARED((NP, 32), F32),
            pltpu.SemaphoreType.DMA,
        ],
        compiler_params=pltpu.CompilerParams(use_tc_tiling_on_sc=False, needs_layout_passes=False),
    )
    def k(xlc_h, src_h, dst_h, ex_h, r_h, out_h,
          srcv, dstv, exv, rv, abuf, idxb, rows, acc, sem):
        cid = lax.axis_index("c")
        sid = lax.axis_index("s")
        pltpu.sync_copy(src_h.at[sid], srcv)
        pltpu.sync_copy(dst_h.at[sid], dstv)
        pltpu.sync_copy(ex_h.at[pl.ds(sid * RPT, RPT)], exv)
        for k4 in range(16):
            chunk = cid * 16 + k4
            hsel = cid * 2 + (k4 // 8)
            if k4 % 8 == 0:
                pltpu.sync_copy(r_h.at[pl.ds(hsel * NP, NP)], rv)
            def zf(i, _):
                for q in range(2):
                    rows[i, pl.ds(q * 16, 16)] = jnp.zeros((16,), F32)
                return 0
            lax.fori_loop(0, 128, zf, 0)
            for z in range(NPT // 128):
                pltpu.sync_copy(rows, acc.at[pl.ds(sid * NPT + z * 128, 128)])
            plsc.subcore_barrier()
            coff = chunk * NP

            def al(i, _):
                ri = i // 8
                cp = (i % 8) * 16
                d16 = dstv[ri, pl.ds(cp, 16)]
                e16 = exv[ri, hsel, pl.ds(cp, 16)]
                r16 = plsc.load_gather(rv, [d16])
                abuf[ri, pl.ds(cp, 16)] = e16 * r16
                idxb[ri, pl.ds(cp, 16)] = srcv[ri, pl.ds(cp, 16)] + coff
                return 0
            lax.fori_loop(0, RPT * 8, al, 0, unroll=2)

            def bl(jb, _):
                pltpu.async_copy(xlc_h.at[idxb.at[jb]], rows, sem).wait()

                def ml(g, _):
                    a16 = abuf[jb, pl.ds(g * 16, 16)]
                    for i in range(16):
                        e = g * 16 + i
                        a_s = a16[i]
                        for q in range(2):
                            rows[e, pl.ds(q * 16, 16)] = (
                                rows[e, pl.ds(q * 16, 16)] * a_s)
                    return 0
                lax.fori_loop(0, 8, ml, 0)
                pltpu.sync_copy(rows, acc.at[dstv.at[jb]], add=True)
                return 0
            lax.fori_loop(0, RPT, bl, 0)
            plsc.subcore_barrier()
            pltpu.sync_copy(acc.at[pl.ds(sid * NPT, NPT)],
                            out_h.at[chunk, pl.ds(sid * NPT, NPT)])
            plsc.subcore_barrier()

    return k(xlc_flat, src128, dst128, exq, r)


def _edge_stage(xl, xr, xlc, ee, src_p, dst_p, att):
    src16 = src_p.reshape(32, EPW // 16, 16)
    dst16 = dst_p.reshape(32, EPW // 16, 16)
    src128b = src_p.reshape(16, RPT, 128)
    dst128b = dst_p.reshape(16, RPT, 128)
    attf = att.reshape(HC)
    exq, den = _edge_a(xl, xr, ee, src16, dst16, attf)
    r = _recip(den.reshape(2, H, NP))
    return _edge_b(xlc.reshape(32 * NP, 32), src128b, dst128b, exq,
                   r.reshape(H * NP))


def _layer(h, src_p, dst_p, ea_p, w):
    (llw, llb, lrw, lrb, lew, att, cb, gg, gb, pw, pb, lng, lnb) = w
    xl, xr, xlc = _lr_proj(h, llw, llb, lrw, lrb)
    ee = _ee_proj(ea_p, lew)
    out = _edge_stage(xl, xr, xlc, ee, src_p, dst_p, att)
    ps = _stats(out, cb)
    return _post(out, ps, cb, gg, gb, pw, pb, lng, lnb)


def kernel(x, edge_index, edge_attr, emb_w, emb_b, emb_ln_g, emb_ln_b,
           lin_l_w0, lin_l_b0, lin_r_w0, lin_r_b0, lin_e_w0, att0, conv_b0,
           gln_g0, gln_b0, proj_w0, proj_b0, ln_g0, ln_b0,
           lin_l_w1, lin_l_b1, lin_r_w1, lin_r_b1, lin_e_w1, att1, conv_b1,
           gln_g1, gln_b1, proj_w1, proj_b1, ln_g1, ln_b1):
    p = dict(locals())
    src = edge_index[0]
    dst = edge_index[1]
    npad = EP - E - N
    loop = jnp.arange(N, dtype=jnp.int32)
    src_p = jnp.concatenate([src, loop, jnp.arange(npad, dtype=jnp.int32) % N])
    dst_p = jnp.concatenate([dst, loop,
                             N + (jnp.arange(npad, dtype=jnp.int32) % 16)])
    ea_p = jnp.concatenate([edge_attr, jnp.zeros((EP - E, DE), F32)], 0)
    xp = jnp.pad(x, ((0, NP - N), (0, 0)))

    h0 = _embed(xp, emb_w, emb_b, emb_ln_g, emb_ln_b)

    names = ['lin_l_w', 'lin_l_b', 'lin_r_w', 'lin_r_b', 'lin_e_w', 'att',
             'conv_b', 'gln_g', 'gln_b', 'proj_w', 'proj_b', 'ln_g', 'ln_b']
    ws = tuple(jnp.stack([p[n + '0'], p[n + '1']]) for n in names)

    def body(h, w):
        hn = _layer(h, src_p, dst_p, ea_p, w)
        return hn, hn

    _, hs = lax.scan(body, h0, ws, length=2)
    return _max3(h0, hs[0], hs[1])[:N]


# speedup vs baseline: 1.2649x; 1.2649x over previous
import functools

import jax
import jax.numpy as jnp
from jax import lax
from jax.experimental import pallas as pl
from jax.experimental.pallas import tpu as pltpu
from jax.experimental.pallas import tpu_sc as plsc

N = 10000
NP = 10240
E = 160000
EP = 172032
D = 256
DE = 16
H = 4
C = 256
HC = H * C
EPS = 1e-5
F32 = jnp.float32

NB = 16
BN = NP // NB
EB = 84
BE = EP // EB


def _leaky(x, s):
    return jnp.where(x >= 0, x, s * x)


def _embed_body(x_ref, w_ref, b_ref, g_ref, bb_ref, o_ref):
    h = jnp.dot(x_ref[...], w_ref[...].T, preferred_element_type=F32) + b_ref[...]
    mu = h.mean(-1, keepdims=True)
    var = ((h - mu) ** 2).mean(-1, keepdims=True)
    h = (h - mu) / jnp.sqrt(var + EPS) * g_ref[...] + bb_ref[...]
    o_ref[...] = _leaky(h, 0.01)


def _embed(xp, w, b, g, bb):
    return pl.pallas_call(
        _embed_body,
        grid=(NB,),
        in_specs=[
            pl.BlockSpec((BN, D), lambda i: (i, 0)),
            pl.BlockSpec((D, D), lambda i: (0, 0)),
            pl.BlockSpec((1, D), lambda i: (0, 0)),
            pl.BlockSpec((1, D), lambda i: (0, 0)),
            pl.BlockSpec((1, D), lambda i: (0, 0)),
        ],
        out_specs=pl.BlockSpec((BN, D), lambda i: (i, 0)),
        out_shape=jax.ShapeDtypeStruct((NP, D), F32),
    )(xp, w, b.reshape(1, D), g.reshape(1, D), bb.reshape(1, D))


def _lr_body(h_ref, wl_ref, bl_ref, wr_ref, br_ref, xl_ref, xr_ref, xlc_ref):
    h = h_ref[...]
    xl = jnp.dot(h, wl_ref[...].T, preferred_element_type=F32) + bl_ref[...]
    xr = jnp.dot(h, wr_ref[...].T, preferred_element_type=F32) + br_ref[...]
    xl_ref[...] = xl
    xr_ref[...] = xr
    for c in range(32):
        xlc_ref[c] = xl[:, c * 32:(c + 1) * 32]


def _lr_proj(h, wl, bl, wr, br):
    return pl.pallas_call(
        _lr_body,
        grid=(NB,),
        in_specs=[
            pl.BlockSpec((BN, D), lambda i: (i, 0)),
            pl.BlockSpec((HC, D), lambda i: (0, 0)),
            pl.BlockSpec((1, HC), lambda i: (0, 0)),
            pl.BlockSpec((HC, D), lambda i: (0, 0)),
            pl.BlockSpec((1, HC), lambda i: (0, 0)),
        ],
        out_specs=[
            pl.BlockSpec((BN, HC), lambda i: (i, 0)),
            pl.BlockSpec((BN, HC), lambda i: (i, 0)),
            pl.BlockSpec((32, BN, 32), lambda i: (0, i, 0)),
        ],
        out_shape=[
            jax.ShapeDtypeStruct((NP, HC), F32),
            jax.ShapeDtypeStruct((NP, HC), F32),
            jax.ShapeDtypeStruct((32, NP, 32), F32),
        ],
    )(h, wl, bl.reshape(1, HC), wr, br.reshape(1, HC))


def _ee_body(ea_ref, we_ref, o_ref):
    o_ref[...] = jnp.dot(ea_ref[...], we_ref[...].T, preferred_element_type=F32)


def _ee_proj(ea_p, we):
    return pl.pallas_call(
        _ee_body,
        grid=(EB,),
        in_specs=[
            pl.BlockSpec((BE, DE), lambda i: (i, 0)),
            pl.BlockSpec((HC, DE), lambda i: (0, 0)),
        ],
        out_specs=pl.BlockSpec((BE, HC), lambda i: (i, 0)),
        out_shape=jax.ShapeDtypeStruct((EP, HC), F32),
    )(ea_p, we)


def _recip_body(d_ref, o_ref):
    o_ref[...] = 1.0 / (d_ref[0] + d_ref[1] + 1e-16)


def _recip(den_part):
    return pl.pallas_call(
        _recip_body,
        grid=(1,),
        in_specs=[pl.BlockSpec((2, H, NP), lambda i: (0, 0, 0))],
        out_specs=pl.BlockSpec((H, NP), lambda i: (0, 0)),
        out_shape=jax.ShapeDtypeStruct((H, NP), F32),
    )(den_part)


def _stats_body(o_ref, cb_ref, p_ref):
    i = pl.program_id(0)
    rid = lax.broadcasted_iota(jnp.int32, (BN, 32), 0) + i * BN
    s = jnp.zeros((), F32)
    sq = jnp.zeros((), F32)
    for c in range(32):
        v = o_ref[c] + cb_ref[c]
        v = jnp.where(rid < N, v, 0.0)
        s = s + jnp.sum(v)
        sq = sq + jnp.sum(v * v)
    col = lax.broadcasted_iota(jnp.int32, (1, 1, 128), 2)
    p_ref[...] = jnp.where(col == 0, s, jnp.where(col == 1, sq, 0.0))


def _stats(out_cm, cb):
    return pl.pallas_call(
        _stats_body,
        grid=(NB,),
        in_specs=[
            pl.BlockSpec((32, BN, 32), lambda i: (0, i, 0)),
            pl.BlockSpec((32, 1, 32), lambda i: (0, 0, 0)),
        ],
        out_specs=pl.BlockSpec((1, 1, 128), lambda i: (i, 0, 0)),
        out_shape=jax.ShapeDtypeStruct((NB, 1, 128), F32),
    )(out_cm, cb.reshape(32, 1, 32))


def _post_body(o_ref, ps_ref, cb_ref, gg_ref, gb_ref, pw_ref, pb_ref,
               g_ref, b_ref, h_ref):
    cnt = float(N * HC)
    s = jnp.sum(ps_ref[:, :, 0])
    sq = jnp.sum(ps_ref[:, :, 1])
    mu = s / cnt
    var = sq / cnt - mu * mu
    std = jnp.sqrt(var)
    zs = []
    for c in range(32):
        v = o_ref[c] + cb_ref[c]
        z = (v - mu) / (std + EPS) * gg_ref[c] + gb_ref[c]
        zs.append(_leaky(z, 0.01))
    z = jnp.concatenate(zs, axis=-1)
    y = jnp.dot(z, pw_ref[...].T, preferred_element_type=F32) + pb_ref[...]
    mu2 = y.mean(-1, keepdims=True)
    var2 = ((y - mu2) ** 2).mean(-1, keepdims=True)
    y = (y - mu2) / jnp.sqrt(var2 + EPS) * g_ref[...] + b_ref[...]
    h_ref[...] = _leaky(y, 0.01)


def _post(out_cm, ps, cb, gg, gb, pw, pb, g, b):
    return pl.pallas_call(
        _post_body,
        grid=(NB,),
        in_specs=[
            pl.BlockSpec((32, BN, 32), lambda i: (0, i, 0)),
            pl.BlockSpec((NB, 1, 128), lambda i: (0, 0, 0)),
            pl.BlockSpec((32, 1, 32), lambda i: (0, 0, 0)),
            pl.BlockSpec((32, 1, 32), lambda i: (0, 0, 0)),
            pl.BlockSpec((32, 1, 32), lambda i: (0, 0, 0)),
            pl.BlockSpec((D, HC), lambda i: (0, 0)),
            pl.BlockSpec((1, D), lambda i: (0, 0)),
            pl.BlockSpec((1, D), lambda i: (0, 0)),
            pl.BlockSpec((1, D), lambda i: (0, 0)),
        ],
        out_specs=pl.BlockSpec((BN, D), lambda i: (i, 0)),
        out_shape=jax.ShapeDtypeStruct((NP, D), F32),
    )(out_cm, ps, cb.reshape(32, 1, 32), gg.reshape(32, 1, 32),
      gb.reshape(32, 1, 32), pw, pb.reshape(1, D), g.reshape(1, D),
      b.reshape(1, D))


def _max3_body(a_ref, b_ref, c_ref, o_ref):
    o_ref[...] = jnp.maximum(jnp.maximum(a_ref[...], b_ref[...]), c_ref[...])


def _max3(a, b, c):
    return pl.pallas_call(
        _max3_body,
        grid=(NB,),
        in_specs=[pl.BlockSpec((BN, D), lambda i: (i, 0))] * 3,
        out_specs=pl.BlockSpec((BN, D), lambda i: (i, 0)),
        out_shape=jax.ShapeDtypeStruct((NP, D), F32),
    )(a, b, c)


EPW = EP // 32
SBW = EPW // 128
ETB = EP // 16
RPT = ETB // 128
NPT = NP // 16


def _edge_a(xl, xr, ee, src16, dst16, attf):
    mesh = plsc.VectorSubcoreMesh(core_axis_name="c", subcore_axis_name="s")

    @functools.partial(
        pl.kernel,
        out_type=[
            jax.ShapeDtypeStruct((EP // 128, H, 128), F32),
            jax.ShapeDtypeStruct((2 * H * NP,), F32),
        ],
        mesh=mesh,
        scratch_types=[
            pltpu.VMEM((EP // 16 // 32, 16), jnp.int32),
            pltpu.VMEM((EP // 16 // 32, 16), jnp.int32),
            pltpu.VMEM((SBW, 128), jnp.int32),
            pltpu.VMEM((2, 16, HC), F32),
            pltpu.VMEM((2, 16, HC), F32),
            pltpu.VMEM((2, 16, HC), F32),
            pltpu.VMEM((H, 128), F32),
            pltpu.VMEM((HC,), F32),
            pltpu.VMEM((NPT,), F32),
            pltpu.VMEM_SHARED((NP,), F32),
            pltpu.VMEM_SHARED((NP,), F32),
            pltpu.VMEM_SHARED((NP,), F32),
            pltpu.VMEM_SHARED((NP,), F32),
            pltpu.SemaphoreType.DMA,
            pltpu.SemaphoreType.DMA,
            pltpu.SemaphoreType.DMA,
            pltpu.SemaphoreType.DMA,
            pltpu.SemaphoreType.DMA,
            pltpu.SemaphoreType.DMA,
        ],
        compiler_params=pltpu.CompilerParams(use_tc_tiling_on_sc=False, needs_layout_passes=False),
    )
    def k(xl_h, xr_h, ee_h, src_h, dst_h, att_h, ex_h, den_h,
          srcv, dstv, dstv128, xlb, xrb, eeb, exrow, attv, zbuf,
          d0, d1, d2, d3, s0, s1, s2, s3, s4, s5):
        dens = (d0, d1, d2, d3)
        cid = lax.axis_index("c")
        sid = lax.axis_index("s")
        wid = sid * 2 + cid
        pltpu.sync_copy(src_h.at[wid], srcv)
        pltpu.sync_copy(dst_h.at[wid], dstv)
        pltpu.sync_copy(att_h, attv)

        def df(i, _):
            dstv128[i // 8, pl.ds((i % 8) * 16, 16)] = dstv[i]
            return 0
        lax.fori_loop(0, EP // 16 // 32, df, 0)

        def zf(i, _):
            zbuf[pl.ds(i * 16, 16)] = jnp.zeros((16,), F32)
            return 0
        lax.fori_loop(0, NPT // 16, zf, 0)
        for dh in dens:
            pltpu.sync_copy(zbuf, dh.at[pl.ds(sid * NPT, NPT)])
        plsc.subcore_barrier()

        NBLK = EPW // 16
        bufs = {0: (xlb.at[0], xrb.at[0], eeb.at[0], s0, s1, s2),
                1: (xlb.at[1], xrb.at[1], eeb.at[1], s3, s4, s5)}

        def fire(b, bset):
            xb, rb, eb, sa, sbm, sc = bset
            bc = jnp.minimum(b, NBLK - 1)
            pltpu.async_copy(xl_h.at[srcv.at[bc]], xb, sa)
            pltpu.async_copy(xr_h.at[dstv.at[bc]], rb, sbm)
            pltpu.async_copy(ee_h.at[pl.ds(wid * EPW + bc * 16, 16)], eb, sc)

        def wait3(bset):
            xb, rb, eb, sa, sbm, sc = bset
            pltpu.make_async_copy(xl_h.at[srcv.at[0]], xb, sa).wait()
            pltpu.make_async_copy(xr_h.at[dstv.at[0]], rb, sbm).wait()
            pltpu.make_async_copy(ee_h.at[pl.ds(0, 16)], eb, sc).wait()

        def compute(b, bset):
            xb, rb, eb = bset[0], bset[1], bset[2]
            sb = b % 8
            for h in range(H):
                def cl(g, acc, h=h):
                    row16 = lax.iota(jnp.int32, 16)
                    c0 = h * C + g * 16
                    att16 = attv[pl.ds(c0, 16)]
                    for jj in range(16):
                        colv = jnp.full((16,), jj, jnp.int32) + c0
                        xlv = plsc.load_gather(xb, [row16, colv])
                        xrv = plsc.load_gather(rb, [row16, colv])
                        eev = plsc.load_gather(eb, [row16, colv])
                        z = xlv + xrv + eev
                        m = jnp.maximum(z, 0.2 * z)
                        acc = acc + att16[jj] * m
                    return acc
                acc = lax.fori_loop(0, C // 16, cl, jnp.zeros((16,), F32))
                exrow[h, pl.ds(sb * 16, 16)] = jnp.exp(acc)

            @pl.when(sb == 7)
            def _flush():
                S = b // 8
                pltpu.sync_copy(exrow, ex_h.at[wid * SBW + S])
                for h in range(H):
                    pltpu.sync_copy(exrow.at[h], dens[h].at[dstv128.at[S]],
                                    add=True)

        fire(0, bufs[0])

        def pipe(i2, _):
            b0 = i2 * 2
            fire(b0 + 1, bufs[1])
            wait3(bufs[0])
            compute(b0, bufs[0])
            fire(b0 + 2, bufs[0])
            wait3(bufs[1])
            compute(b0 + 1, bufs[1])
            return 0
        lax.fori_loop(0, NBLK // 2, pipe, 0)
        wait3(bufs[0])
        plsc.subcore_barrier()
        for h in range(H):
            off = cid * (H * NP) + h * NP + sid * NPT
            pltpu.sync_copy(dens[h].at[pl.ds(sid * NPT, NPT)],
                            den_h.at[pl.ds(off, NPT)])

    return k(xl, xr, ee, src16, dst16, attf)


def _edge_b(xlc_flat, src128, dst128, exq, r):
    mesh = plsc.VectorSubcoreMesh(core_axis_name="c", subcore_axis_name="s")

    @functools.partial(
        pl.kernel,
        out_type=jax.ShapeDtypeStruct((32, NP, 32), F32),
        mesh=mesh,
        scratch_types=[
            pltpu.VMEM((RPT, 128), jnp.int32),
            pltpu.VMEM((RPT, 128), jnp.int32),
            pltpu.VMEM((RPT, H, 128), F32),
            pltpu.VMEM((NP,), F32),
            pltpu.VMEM((RPT, 128), F32),
            pltpu.VMEM((RPT, 128), jnp.int32),
            pltpu.VMEM((128, 32), F32),
            pltpu.VMEM_SHARED((NP, 32), F32),
            pltpu.SemaphoreType.DMA,
        ],
        compiler_params=pltpu.CompilerParams(use_tc_tiling_on_sc=False, needs_layout_passes=False),
    )
    def k(xlc_h, src_h, dst_h, ex_h, r_h, out_h,
          srcv, dstv, exv, rv, abuf, idxb, rows, acc, sem):
        cid = lax.axis_index("c")
        sid = lax.axis_index("s")
        pltpu.sync_copy(src_h.at[sid], srcv)
        pltpu.sync_copy(dst_h.at[sid], dstv)
        pltpu.sync_copy(ex_h.at[pl.ds(sid * RPT, RPT)], exv)
        for k4 in range(16):
            chunk = cid * 16 + k4
            hsel = cid * 2 + (k4 // 8)
            if k4 % 8 == 0:
                pltpu.sync_copy(r_h.at[pl.ds(hsel * NP, NP)], rv)
            def zf(i, _):
                for q in range(2):
                    rows[i, pl.ds(q * 16, 16)] = jnp.zeros((16,), F32)
                return 0
            lax.fori_loop(0, 128, zf, 0)
            for z in range(NPT // 128):
                pltpu.sync_copy(rows, acc.at[pl.ds(sid * NPT + z * 128, 128)])
            plsc.subcore_barrier()
            coff = chunk * NP

            def al(i, _):
                ri = i // 8
                cp = (i % 8) * 16
                d16 = dstv[ri, pl.ds(cp, 16)]
                e16 = exv[ri, hsel, pl.ds(cp, 16)]
                r16 = plsc.load_gather(rv, [d16])
                abuf[ri, pl.ds(cp, 16)] = e16 * r16
                idxb[ri, pl.ds(cp, 16)] = srcv[ri, pl.ds(cp, 16)] + coff
                return 0
            lax.fori_loop(0, RPT * 8, al, 0, unroll=2)

            def bl(jb, _):
                pltpu.async_copy(xlc_h.at[idxb.at[jb]], rows, sem).wait()

                def ml(g, _):
                    a16 = abuf[jb, pl.ds(g * 16, 16)]
                    for i in range(16):
                        e = g * 16 + i
                        a_s = a16[i]
                        for q in range(2):
                            rows[e, pl.ds(q * 16, 16)] = (
                                rows[e, pl.ds(q * 16, 16)] * a_s)
                    return 0
                lax.fori_loop(0, 8, ml, 0)
                pltpu.sync_copy(rows, acc.at[dstv.at[jb]], add=True)
                return 0
            lax.fori_loop(0, RPT, bl, 0)
            plsc.subcore_barrier()
            pltpu.sync_copy(acc.at[pl.ds(sid * NPT, NPT)],
                            out_h.at[chunk, pl.ds(sid * NPT, NPT)])
            plsc.subcore_barrier()

    return k(xlc_flat, src128, dst128, exq, r)


def _edge_stage(xl, xr, xlc, ee, src_p, dst_p, att):
    src16 = src_p.reshape(32, EPW // 16, 16)
    dst16 = dst_p.reshape(32, EPW // 16, 16)
    src128b = src_p.reshape(16, RPT, 128)
    dst128b = dst_p.reshape(16, RPT, 128)
    attf = att.reshape(HC)
    exq, den = _edge_a(xl, xr, ee, src16, dst16, attf)
    r = _recip(den.reshape(2, H, NP))
    return _edge_b(xlc.reshape(32 * NP, 32), src128b, dst128b, exq,
                   r.reshape(H * NP))


def _layer(h, src_p, dst_p, ea_p, w):
    (llw, llb, lrw, lrb, lew, att, cb, gg, gb, pw, pb, lng, lnb) = w
    xl, xr, xlc = _lr_proj(h, llw, llb, lrw, lrb)
    ee = _ee_proj(ea_p, lew)
    out = _edge_stage(xl, xr, xlc, ee, src_p, dst_p, att)
    ps = _stats(out, cb)
    return _post(out, ps, cb, gg, gb, pw, pb, lng, lnb)


def kernel(x, edge_index, edge_attr, emb_w, emb_b, emb_ln_g, emb_ln_b,
           lin_l_w0, lin_l_b0, lin_r_w0, lin_r_b0, lin_e_w0, att0, conv_b0,
           gln_g0, gln_b0, proj_w0, proj_b0, ln_g0, ln_b0,
           lin_l_w1, lin_l_b1, lin_r_w1, lin_r_b1, lin_e_w1, att1, conv_b1,
           gln_g1, gln_b1, proj_w1, proj_b1, ln_g1, ln_b1):
    p = dict(locals())
    src = edge_index[0]
    dst = edge_index[1]
    npad = EP - E - N
    loop = jnp.arange(N, dtype=jnp.int32)
    src_p = jnp.concatenate([src, loop, jnp.arange(npad, dtype=jnp.int32) % N])
    dst_p = jnp.concatenate([dst, loop,
                             N + (jnp.arange(npad, dtype=jnp.int32) % 16)])
    ea_p = jnp.concatenate([edge_attr, jnp.zeros((EP - E, DE), F32)], 0)
    xp = jnp.pad(x, ((0, NP - N), (0, 0)))

    h0 = _embed(xp, emb_w, emb_b, emb_ln_g, emb_ln_b)

    names = ['lin_l_w', 'lin_l_b', 'lin_r_w', 'lin_r_b', 'lin_e_w', 'att',
             'conv_b', 'gln_g', 'gln_b', 'proj_w', 'proj_b', 'ln_g', 'ln_b']
    ws = tuple(jnp.stack([p[n + '0'], p[n + '1']]) for n in names)

    def body(h, w):
        hn = _layer(h, src_p, dst_p, ea_p, w)
        return hn, hn

    _, hs = lax.scan(body, h0, ws, length=2)
    return _max3(h0, hs[0], hs[1])[:N]


# speedup vs baseline: 3.0769x; 2.4324x over previous
import functools

import jax
import jax.numpy as jnp
from jax import lax
from jax.experimental import pallas as pl
from jax.experimental.pallas import tpu as pltpu
from jax.experimental.pallas import tpu_sc as plsc

N = 10000
NP = 10240
E = 160000
EP = 172032
D = 256
DE = 16
H = 4
C = 256
HC = H * C
EPS = 1e-5
F32 = jnp.float32

NB = 16
BN = NP // NB
EB = 84
BE = EP // EB


def _leaky(x, s):
    return jnp.where(x >= 0, x, s * x)


def _embed_body(x_ref, w_ref, b_ref, g_ref, bb_ref, o_ref):
    h = jnp.dot(x_ref[...], w_ref[...].T, preferred_element_type=F32) + b_ref[...]
    mu = h.mean(-1, keepdims=True)
    var = ((h - mu) ** 2).mean(-1, keepdims=True)
    h = (h - mu) / jnp.sqrt(var + EPS) * g_ref[...] + bb_ref[...]
    o_ref[...] = _leaky(h, 0.01)


def _embed(xp, w, b, g, bb):
    return pl.pallas_call(
        _embed_body,
        grid=(NB,),
        in_specs=[
            pl.BlockSpec((BN, D), lambda i: (i, 0)),
            pl.BlockSpec((D, D), lambda i: (0, 0)),
            pl.BlockSpec((1, D), lambda i: (0, 0)),
            pl.BlockSpec((1, D), lambda i: (0, 0)),
            pl.BlockSpec((1, D), lambda i: (0, 0)),
        ],
        out_specs=pl.BlockSpec((BN, D), lambda i: (i, 0)),
        out_shape=jax.ShapeDtypeStruct((NP, D), F32),
    )(xp, w, b.reshape(1, D), g.reshape(1, D), bb.reshape(1, D))


def _lr_body(h_ref, wl_ref, bl_ref, wr_ref, br_ref, xl_ref, xr_ref, xlc_ref):
    h = h_ref[...]
    xl = jnp.dot(h, wl_ref[...].T, preferred_element_type=F32) + bl_ref[...]
    xr = jnp.dot(h, wr_ref[...].T, preferred_element_type=F32) + br_ref[...]
    xl_ref[...] = xl
    xr_ref[...] = xr
    for c in range(32):
        xlc_ref[c] = xl[:, c * 32:(c + 1) * 32]


def _lr_proj(h, wl, bl, wr, br):
    return pl.pallas_call(
        _lr_body,
        grid=(NB,),
        in_specs=[
            pl.BlockSpec((BN, D), lambda i: (i, 0)),
            pl.BlockSpec((HC, D), lambda i: (0, 0)),
            pl.BlockSpec((1, HC), lambda i: (0, 0)),
            pl.BlockSpec((HC, D), lambda i: (0, 0)),
            pl.BlockSpec((1, HC), lambda i: (0, 0)),
        ],
        out_specs=[
            pl.BlockSpec((BN, HC), lambda i: (i, 0)),
            pl.BlockSpec((BN, HC), lambda i: (i, 0)),
            pl.BlockSpec((32, BN, 32), lambda i: (0, i, 0)),
        ],
        out_shape=[
            jax.ShapeDtypeStruct((NP, HC), F32),
            jax.ShapeDtypeStruct((NP, HC), F32),
            jax.ShapeDtypeStruct((32, NP, 32), F32),
        ],
    )(h, wl, bl.reshape(1, HC), wr, br.reshape(1, HC))


def _ee_body(ea_ref, we_ref, o_ref):
    o_ref[...] = jnp.dot(ea_ref[...], we_ref[...].T, preferred_element_type=F32)


def _ee_proj(ea_p, we):
    return pl.pallas_call(
        _ee_body,
        grid=(EB,),
        in_specs=[
            pl.BlockSpec((BE, DE), lambda i: (i, 0)),
            pl.BlockSpec((HC, DE), lambda i: (0, 0)),
        ],
        out_specs=pl.BlockSpec((BE, HC), lambda i: (i, 0)),
        out_shape=jax.ShapeDtypeStruct((EP, HC), F32),
    )(ea_p, we)


def _recip_body(d_ref, o_ref):
    o_ref[...] = 1.0 / (d_ref[0] + d_ref[1] + 1e-16)


def _recip(den_part):
    return pl.pallas_call(
        _recip_body,
        grid=(1,),
        in_specs=[pl.BlockSpec((2, H, NP), lambda i: (0, 0, 0))],
        out_specs=pl.BlockSpec((H, NP), lambda i: (0, 0)),
        out_shape=jax.ShapeDtypeStruct((H, NP), F32),
    )(den_part)


def _stats_body(o_ref, cb_ref, p_ref):
    i = pl.program_id(0)
    rid = lax.broadcasted_iota(jnp.int32, (BN, 32), 0) + i * BN
    s = jnp.zeros((), F32)
    sq = jnp.zeros((), F32)
    for c in range(32):
        v = o_ref[c] + cb_ref[c]
        v = jnp.where(rid < N, v, 0.0)
        s = s + jnp.sum(v)
        sq = sq + jnp.sum(v * v)
    col = lax.broadcasted_iota(jnp.int32, (1, 1, 128), 2)
    p_ref[...] = jnp.where(col == 0, s, jnp.where(col == 1, sq, 0.0))


def _stats(out_cm, cb):
    return pl.pallas_call(
        _stats_body,
        grid=(NB,),
        in_specs=[
            pl.BlockSpec((32, BN, 32), lambda i: (0, i, 0)),
            pl.BlockSpec((32, 1, 32), lambda i: (0, 0, 0)),
        ],
        out_specs=pl.BlockSpec((1, 1, 128), lambda i: (i, 0, 0)),
        out_shape=jax.ShapeDtypeStruct((NB, 1, 128), F32),
    )(out_cm, cb.reshape(32, 1, 32))


def _post_body(o_ref, ps_ref, cb_ref, gg_ref, gb_ref, pw_ref, pb_ref,
               g_ref, b_ref, h_ref):
    cnt = float(N * HC)
    s = jnp.sum(ps_ref[:, :, 0])
    sq = jnp.sum(ps_ref[:, :, 1])
    mu = s / cnt
    var = sq / cnt - mu * mu
    std = jnp.sqrt(var)
    zs = []
    for c in range(32):
        v = o_ref[c] + cb_ref[c]
        z = (v - mu) / (std + EPS) * gg_ref[c] + gb_ref[c]
        zs.append(_leaky(z, 0.01))
    z = jnp.concatenate(zs, axis=-1)
    y = jnp.dot(z, pw_ref[...].T, preferred_element_type=F32) + pb_ref[...]
    mu2 = y.mean(-1, keepdims=True)
    var2 = ((y - mu2) ** 2).mean(-1, keepdims=True)
    y = (y - mu2) / jnp.sqrt(var2 + EPS) * g_ref[...] + b_ref[...]
    h_ref[...] = _leaky(y, 0.01)


def _post(out_cm, ps, cb, gg, gb, pw, pb, g, b):
    return pl.pallas_call(
        _post_body,
        grid=(NB,),
        in_specs=[
            pl.BlockSpec((32, BN, 32), lambda i: (0, i, 0)),
            pl.BlockSpec((NB, 1, 128), lambda i: (0, 0, 0)),
            pl.BlockSpec((32, 1, 32), lambda i: (0, 0, 0)),
            pl.BlockSpec((32, 1, 32), lambda i: (0, 0, 0)),
            pl.BlockSpec((32, 1, 32), lambda i: (0, 0, 0)),
            pl.BlockSpec((D, HC), lambda i: (0, 0)),
            pl.BlockSpec((1, D), lambda i: (0, 0)),
            pl.BlockSpec((1, D), lambda i: (0, 0)),
            pl.BlockSpec((1, D), lambda i: (0, 0)),
        ],
        out_specs=pl.BlockSpec((BN, D), lambda i: (i, 0)),
        out_shape=jax.ShapeDtypeStruct((NP, D), F32),
    )(out_cm, ps, cb.reshape(32, 1, 32), gg.reshape(32, 1, 32),
      gb.reshape(32, 1, 32), pw, pb.reshape(1, D), g.reshape(1, D),
      b.reshape(1, D))


def _max3_body(a_ref, b_ref, c_ref, o_ref):
    o_ref[...] = jnp.maximum(jnp.maximum(a_ref[...], b_ref[...]), c_ref[...])


def _max3(a, b, c):
    return pl.pallas_call(
        _max3_body,
        grid=(NB,),
        in_specs=[pl.BlockSpec((BN, D), lambda i: (i, 0))] * 3,
        out_specs=pl.BlockSpec((BN, D), lambda i: (i, 0)),
        out_shape=jax.ShapeDtypeStruct((NP, D), F32),
    )(a, b, c)


EPW = EP // 32
SBW = EPW // 128
ETB = EP // 16
RPT = ETB // 128
NPT = NP // 16


def _edge_a(xl, xr, ee, src16, dst16, attf):
    mesh = plsc.VectorSubcoreMesh(core_axis_name="c", subcore_axis_name="s")

    @functools.partial(
        pl.kernel,
        out_type=[
            jax.ShapeDtypeStruct((EP // 128, H, 128), F32),
            jax.ShapeDtypeStruct((2 * H * NP,), F32),
        ],
        mesh=mesh,
        scratch_types=[
            pltpu.VMEM((EP // 16 // 32, 16), jnp.int32),
            pltpu.VMEM((EP // 16 // 32, 16), jnp.int32),
            pltpu.VMEM((SBW, 128), jnp.int32),
            pltpu.VMEM((2, 16, HC), F32),
            pltpu.VMEM((2, 16, HC), F32),
            pltpu.VMEM((2, 16, HC), F32),
            pltpu.VMEM((H, 128), F32),
            pltpu.VMEM((HC,), F32),
            pltpu.VMEM((NPT,), F32),
            pltpu.VMEM_SHARED((NP,), F32),
            pltpu.VMEM_SHARED((NP,), F32),
            pltpu.VMEM_SHARED((NP,), F32),
            pltpu.VMEM_SHARED((NP,), F32),
            pltpu.SemaphoreType.DMA,
            pltpu.SemaphoreType.DMA,
            pltpu.SemaphoreType.DMA,
            pltpu.SemaphoreType.DMA,
            pltpu.SemaphoreType.DMA,
            pltpu.SemaphoreType.DMA,
        ],
        compiler_params=pltpu.CompilerParams(use_tc_tiling_on_sc=False, needs_layout_passes=False),
    )
    def k(xl_h, xr_h, ee_h, src_h, dst_h, att_h, ex_h, den_h,
          srcv, dstv, dstv128, xlb, xrb, eeb, exrow, attv, zbuf,
          d0, d1, d2, d3, s0, s1, s2, s3, s4, s5):
        dens = (d0, d1, d2, d3)
        cid = lax.axis_index("c")
        sid = lax.axis_index("s")
        wid = sid * 2 + cid
        pltpu.sync_copy(src_h.at[wid], srcv)
        pltpu.sync_copy(dst_h.at[wid], dstv)
        pltpu.sync_copy(att_h, attv)

        def df(i, _):
            dstv128[i // 8, pl.ds((i % 8) * 16, 16)] = dstv[i]
            return 0
        lax.fori_loop(0, EP // 16 // 32, df, 0)

        def zf(i, _):
            zbuf[pl.ds(i * 16, 16)] = jnp.zeros((16,), F32)
            return 0
        lax.fori_loop(0, NPT // 16, zf, 0)
        for dh in dens:
            pltpu.sync_copy(zbuf, dh.at[pl.ds(sid * NPT, NPT)])
        plsc.subcore_barrier()

        NBLK = EPW // 16
        bufs = {0: (xlb.at[0], xrb.at[0], eeb.at[0], s0, s1, s2),
                1: (xlb.at[1], xrb.at[1], eeb.at[1], s3, s4, s5)}

        def fire(b, bset):
            xb, rb, eb, sa, sbm, sc = bset
            bc = jnp.minimum(b, NBLK - 1)
            pltpu.async_copy(xl_h.at[srcv.at[bc]], xb, sa)
            pltpu.async_copy(xr_h.at[dstv.at[bc]], rb, sbm)
            pltpu.async_copy(ee_h.at[pl.ds(wid * EPW + bc * 16, 16)], eb, sc)

        def wait3(bset):
            xb, rb, eb, sa, sbm, sc = bset
            pltpu.make_async_copy(xl_h.at[srcv.at[0]], xb, sa).wait()
            pltpu.make_async_copy(xr_h.at[dstv.at[0]], rb, sbm).wait()
            pltpu.make_async_copy(ee_h.at[pl.ds(0, 16)], eb, sc).wait()

        def compute(b, bset):
            xb, rb, eb = bset[0], bset[1], bset[2]
            sb = b % 8
            for h in range(H):
                def cl(g, acc, h=h):
                    row16 = lax.iota(jnp.int32, 16)
                    c0 = h * C + g * 16
                    att16 = attv[pl.ds(c0, 16)]
                    for jj in range(16):
                        idxm = jnp.bitwise_and(row16 + jj, 15)
                        colv = idxm + c0
                        xlv = plsc.load_gather(xb, [row16, colv])
                        xrv = plsc.load_gather(rb, [row16, colv])
                        eev = plsc.load_gather(eb, [row16, colv])
                        z = xlv + xrv + eev
                        m = jnp.maximum(z, 0.2 * z)
                        attg = att16[idxm]
                        acc = acc + attg * m
                    return acc
                acc = lax.fori_loop(0, C // 16, cl, jnp.zeros((16,), F32))
                exrow[h, pl.ds(sb * 16, 16)] = jnp.exp(acc)

            @pl.when(sb == 7)
            def _flush():
                S = b // 8
                pltpu.sync_copy(exrow, ex_h.at[wid * SBW + S])
                for h in range(H):
                    pltpu.sync_copy(exrow.at[h], dens[h].at[dstv128.at[S]],
                                    add=True)

        fire(0, bufs[0])

        def pipe(i2, _):
            b0 = i2 * 2
            fire(b0 + 1, bufs[1])
            wait3(bufs[0])
            compute(b0, bufs[0])
            fire(b0 + 2, bufs[0])
            wait3(bufs[1])
            compute(b0 + 1, bufs[1])
            return 0
        lax.fori_loop(0, NBLK // 2, pipe, 0)
        wait3(bufs[0])
        plsc.subcore_barrier()
        for h in range(H):
            off = cid * (H * NP) + h * NP + sid * NPT
            pltpu.sync_copy(dens[h].at[pl.ds(sid * NPT, NPT)],
                            den_h.at[pl.ds(off, NPT)])

    return k(xl, xr, ee, src16, dst16, attf)


def _edge_b(xlc_flat, src128, dst128, exq, r):
    mesh = plsc.VectorSubcoreMesh(core_axis_name="c", subcore_axis_name="s")

    @functools.partial(
        pl.kernel,
        out_type=jax.ShapeDtypeStruct((32, NP, 32), F32),
        mesh=mesh,
        scratch_types=[
            pltpu.VMEM((RPT, 128), jnp.int32),
            pltpu.VMEM((RPT, 128), jnp.int32),
            pltpu.VMEM((RPT, H, 128), F32),
            pltpu.VMEM((NP,), F32),
            pltpu.VMEM((RPT, 128), F32),
            pltpu.VMEM((RPT, 128), jnp.int32),
            pltpu.VMEM((128, 32), F32),
            pltpu.VMEM_SHARED((NP, 32), F32),
            pltpu.SemaphoreType.DMA,
        ],
        compiler_params=pltpu.CompilerParams(use_tc_tiling_on_sc=False, needs_layout_passes=False),
    )
    def k(xlc_h, src_h, dst_h, ex_h, r_h, out_h,
          srcv, dstv, exv, rv, abuf, idxb, rows, acc, sem):
        cid = lax.axis_index("c")
        sid = lax.axis_index("s")
        pltpu.sync_copy(src_h.at[sid], srcv)
        pltpu.sync_copy(dst_h.at[sid], dstv)
        pltpu.sync_copy(ex_h.at[pl.ds(sid * RPT, RPT)], exv)
        for k4 in range(16):
            chunk = cid * 16 + k4
            hsel = cid * 2 + (k4 // 8)
            if k4 % 8 == 0:
                pltpu.sync_copy(r_h.at[pl.ds(hsel * NP, NP)], rv)
            def zf(i, _):
                for q in range(2):
                    rows[i, pl.ds(q * 16, 16)] = jnp.zeros((16,), F32)
                return 0
            lax.fori_loop(0, 128, zf, 0)
            for z in range(NPT // 128):
                pltpu.sync_copy(rows, acc.at[pl.ds(sid * NPT + z * 128, 128)])
            plsc.subcore_barrier()
            coff = chunk * NP

            def al(i, _):
                ri = i // 8
                cp = (i % 8) * 16
                d16 = dstv[ri, pl.ds(cp, 16)]
                e16 = exv[ri, hsel, pl.ds(cp, 16)]
                r16 = plsc.load_gather(rv, [d16])
                abuf[ri, pl.ds(cp, 16)] = e16 * r16
                idxb[ri, pl.ds(cp, 16)] = srcv[ri, pl.ds(cp, 16)] + coff
                return 0
            lax.fori_loop(0, RPT * 8, al, 0, unroll=2)

            def bl(jb, _):
                pltpu.async_copy(xlc_h.at[idxb.at[jb]], rows, sem).wait()

                def ml(g, _):
                    a16 = abuf[jb, pl.ds(g * 16, 16)]
                    for i in range(16):
                        e = g * 16 + i
                        a_s = a16[i]
                        for q in range(2):
                            rows[e, pl.ds(q * 16, 16)] = (
                                rows[e, pl.ds(q * 16, 16)] * a_s)
                    return 0
                lax.fori_loop(0, 8, ml, 0)
                pltpu.sync_copy(rows, acc.at[dstv.at[jb]], add=True)
                return 0
            lax.fori_loop(0, RPT, bl, 0)
            plsc.subcore_barrier()
            pltpu.sync_copy(acc.at[pl.ds(sid * NPT, NPT)],
                            out_h.at[chunk, pl.ds(sid * NPT, NPT)])
            plsc.subcore_barrier()

    return k(xlc_flat, src128, dst128, exq, r)


def _edge_stage(xl, xr, xlc, ee, src_p, dst_p, att):
    src16 = src_p.reshape(32, EPW // 16, 16)
    dst16 = dst_p.reshape(32, EPW // 16, 16)
    src128b = src_p.reshape(16, RPT, 128)
    dst128b = dst_p.reshape(16, RPT, 128)
    attf = att.reshape(HC)
    exq, den = _edge_a(xl, xr, ee, src16, dst16, attf)
    r = _recip(den.reshape(2, H, NP))
    return _edge_b(xlc.reshape(32 * NP, 32), src128b, dst128b, exq,
                   r.reshape(H * NP))


def _layer(h, src_p, dst_p, ea_p, w):
    (llw, llb, lrw, lrb, lew, att, cb, gg, gb, pw, pb, lng, lnb) = w
    xl, xr, xlc = _lr_proj(h, llw, llb, lrw, lrb)
    ee = _ee_proj(ea_p, lew)
    out = _edge_stage(xl, xr, xlc, ee, src_p, dst_p, att)
    ps = _stats(out, cb)
    return _post(out, ps, cb, gg, gb, pw, pb, lng, lnb)


def kernel(x, edge_index, edge_attr, emb_w, emb_b, emb_ln_g, emb_ln_b,
           lin_l_w0, lin_l_b0, lin_r_w0, lin_r_b0, lin_e_w0, att0, conv_b0,
           gln_g0, gln_b0, proj_w0, proj_b0, ln_g0, ln_b0,
           lin_l_w1, lin_l_b1, lin_r_w1, lin_r_b1, lin_e_w1, att1, conv_b1,
           gln_g1, gln_b1, proj_w1, proj_b1, ln_g1, ln_b1):
    p = dict(locals())
    src = edge_index[0]
    dst = edge_index[1]
    npad = EP - E - N
    loop = jnp.arange(N, dtype=jnp.int32)
    src_p = jnp.concatenate([src, loop, jnp.arange(npad, dtype=jnp.int32) % N])
    dst_p = jnp.concatenate([dst, loop,
                             N + (jnp.arange(npad, dtype=jnp.int32) % 16)])
    ea_p = jnp.concatenate([edge_attr, jnp.zeros((EP - E, DE), F32)], 0)
    xp = jnp.pad(x, ((0, NP - N), (0, 0)))

    h0 = _embed(xp, emb_w, emb_b, emb_ln_g, emb_ln_b)

    names = ['lin_l_w', 'lin_l_b', 'lin_r_w', 'lin_r_b', 'lin_e_w', 'att',
             'conv_b', 'gln_g', 'gln_b', 'proj_w', 'proj_b', 'ln_g', 'ln_b']
    ws = tuple(jnp.stack([p[n + '0'], p[n + '1']]) for n in names)

    def body(h, w):
        hn = _layer(h, src_p, dst_p, ea_p, w)
        return hn, hn

    _, hs = lax.scan(body, h0, ws, length=2)
    return _max3(h0, hs[0], hs[1])[:N]
